# baseline (speedup 1.0000x reference)
import functools

import jax
import jax.numpy as jnp
from jax import lax
from jax.experimental import pallas as pl
from jax.experimental.pallas import tpu as pltpu
from jax.experimental.pallas import tpu_sc as plsc

NC = 2
NS = 16
NW = NC * NS
S = 200
SPC = 2
T = S // SPC
NBUF = 4

V = 1000000
LB = 1024
TC_GRID = (V + LB - 1) // LB


def _tc_relayout_body(x_ref, o_ref):
    t = x_ref[...].T.reshape(LB // 4, 4, 32)
    for j in range(4):
        o_ref[:, j * 32:(j + 1) * 32] = t[:, j, :]


def _table_relayout(emb_t):
    return pl.pallas_call(
        _tc_relayout_body,
        grid=(TC_GRID,),
        in_specs=[pl.BlockSpec((32, LB), lambda i: (0, i))],
        out_specs=pl.BlockSpec((LB // 4, 128), lambda i: (i, 0)),
        out_shape=jax.ShapeDtypeStruct((V // 4, 128), jnp.float32),
    )(emb_t)


def _make_kernel():
    mesh = plsc.VectorSubcoreMesh(core_axis_name="c", subcore_axis_name="s")

    @functools.partial(
        pl.kernel,
        out_type=jax.ShapeDtypeStruct((S, 4, NW, 1024), jnp.float32),
        mesh=mesh,
        scratch_types=[
            pltpu.VMEM((25, 1024), jnp.int32),
            pltpu.VMEM((NBUF, 128 * SPC, 32), jnp.float32),
            pltpu.VMEM((NBUF, SPC, 4, 1024), jnp.float32),
            pltpu.SemaphoreType.DMA((NBUF,)),
            pltpu.SemaphoreType.DMA((NBUF,)),
            pltpu.SemaphoreType.DMA((NBUF,)),
        ],
        compiler_params=pltpu.CompilerParams(
            use_tc_tiling_on_sc=False, needs_layout_passes=False),
    )
    def run(x5, l4, emb, out5, lv, g, xb, s_ld, s_g, s_st):
        w = lax.axis_index("s") * NC + lax.axis_index("c")
        pltpu.sync_copy(l4.at[:, w], lv)

        def idx_ref(t):
            return lv.at[t // 4, pl.ds((t % 4) * 256, 256)]

        def ld_start(t, b):
            pltpu.async_copy(x5.at[pl.ds(t * SPC, SPC), :, w], xb.at[b],
                             s_ld.at[b])

        def ld_wait(t, b):
            pltpu.make_async_copy(x5.at[pl.ds(t * SPC, SPC), :, w], xb.at[b],
                                  s_ld.at[b]).wait()

        def g_start(t, b):
            pltpu.async_copy(emb.at[idx_ref(t)], g.at[b], s_g.at[b])

        def g_wait(t, b):
            pltpu.make_async_copy(emb.at[idx_ref(t)], g.at[b],
                                  s_g.at[b]).wait()

        def st_start(t, b):
            pltpu.async_copy(xb.at[b], out5.at[pl.ds(t * SPC, SPC), :, w],
                             s_st.at[b])

        def st_wait(t, b):
            pltpu.make_async_copy(xb.at[b], out5.at[pl.ds(t * SPC, SPC), :, w],
                                  s_st.at[b]).wait()

        rows = [jnp.arange(16, dtype=jnp.int32) + 16 * blk for blk in range(8)]

        def compute(b):
            gb = g.at[b]
            xbb = xb.at[b]

            @plsc.parallel_loop(0, 32 * SPC, unroll=8)
            def _(q):
                k = q // 32
                d = q % 32
                dcol = jnp.full((16,), d, dtype=jnp.int32)
                dh = d // 8
                off = (d % 8) * 128
                base = k * 128
                vals = [plsc.load_gather(gb, [base + rows[blk], dcol])
                        for blk in range(8)]
                for blk in range(8):
                    plsc.addupdate(
                        xbb.at[k, dh, pl.ds(off + blk * 16, 16)], vals[blk])

        for t in range(NBUF):
            ld_start(t, t)
            g_start(t, t)
        for t in (0, 1):
            g_wait(t, t)
            ld_wait(t, t)
            compute(t)
            st_start(t, t)

        @pl.loop(2, T - 2, step=4)
        def _(j):
            for k in range(4):
                t = j + k
                b = (2 + k) % 4
                bn = (b + 2) % 4
                st_wait(t - 2, bn)
                ld_start(t + 2, bn)
                g_start(t + 2, bn)
                g_wait(t, b)
                ld_wait(t, b)
                compute(b)
                st_start(t, b)

        for t in (T - 2, T - 1):
            b = t % 4
            st_wait(t - 2, (t - 2) % 4)
            g_wait(t, b)
            ld_wait(t, b)
            compute(b)
            st_start(t, b)
        st_wait(T - 2, (T - 2) % 4)
        st_wait(T - 1, (T - 1) % 4)

    return run


def kernel(inputs, labels, embedding):
    b, s, d = inputs.shape
    x5 = (inputs.transpose(1, 2, 0).reshape(s, 4, 8, 32, 128)
          .transpose(0, 1, 3, 2, 4).reshape(s, 4, 32, 1024))
    l4 = (labels.astype(jnp.int32).T.reshape(25, 8, 32, 128)
          .transpose(0, 2, 1, 3).reshape(25, 32, 1024))
    emb_rm = _table_relayout(embedding.T).reshape(V, d)
    out5 = _make_kernel()(x5, l4, emb_rm)
    return (out5.reshape(s, 4, 32, 8, 128).transpose(0, 1, 3, 2, 4)
            .reshape(s, d, b).transpose(2, 0, 1))


# speedup vs baseline: 1.4088x; 1.4088x over previous
import functools

import jax
import jax.numpy as jnp
from jax import lax
from jax.experimental import pallas as pl
from jax.experimental.pallas import tpu as pltpu
from jax.experimental.pallas import tpu_sc as plsc

NC = 2
NS = 16
NW = NC * NS
S = 200
SPC = 2
T = S // SPC
NBUF = 4

V = 1000000
LB = 4096
TC_GRID = (V + LB - 1) // LB


def _tc_relayout_body(x_ref, o_ref):
    t = x_ref[...].T.reshape(LB // 4, 4, 32)
    for j in range(4):
        o_ref[:, j * 32:(j + 1) * 32] = t[:, j, :]


def _table_relayout(emb_t):
    return pl.pallas_call(
        _tc_relayout_body,
        grid=(TC_GRID,),
        in_specs=[pl.BlockSpec((32, LB), lambda i: (0, i))],
        out_specs=pl.BlockSpec((LB // 4, 128), lambda i: (i, 0)),
        out_shape=jax.ShapeDtypeStruct((V // 4, 128), jnp.float32),
    )(emb_t)


def _make_kernel():
    mesh = plsc.VectorSubcoreMesh(core_axis_name="c", subcore_axis_name="s")

    @functools.partial(
        pl.kernel,
        out_type=jax.ShapeDtypeStruct((S, 4, NW, 1024), jnp.float32),
        mesh=mesh,
        scratch_types=[
            pltpu.VMEM((25, 1024), jnp.int32),
            pltpu.VMEM((NBUF, 128 * SPC, 32), jnp.float32),
            pltpu.VMEM((NBUF, SPC, 4, 1024), jnp.float32),
            pltpu.SemaphoreType.DMA((NBUF,)),
            pltpu.SemaphoreType.DMA((NBUF,)),
            pltpu.SemaphoreType.DMA((NBUF,)),
        ],
        compiler_params=pltpu.CompilerParams(
            use_tc_tiling_on_sc=False, needs_layout_passes=False),
    )
    def run(x5, l4, emb, out5, lv, g, xb, s_ld, s_g, s_st):
        w = lax.axis_index("s") * NC + lax.axis_index("c")
        pltpu.sync_copy(l4.at[:, w], lv)

        def idx_ref(t):
            return lv.at[t // 4, pl.ds((t % 4) * 256, 256)]

        def ld_start(t, b):
            pltpu.async_copy(x5.at[pl.ds(t * SPC, SPC), :, w], xb.at[b],
                             s_ld.at[b])

        def ld_wait(t, b):
            pltpu.make_async_copy(x5.at[pl.ds(t * SPC, SPC), :, w], xb.at[b],
                                  s_ld.at[b]).wait()

        def g_start(t, b):
            pltpu.async_copy(emb.at[idx_ref(t)], g.at[b], s_g.at[b])

        def g_wait(t, b):
            pltpu.make_async_copy(emb.at[idx_ref(t)], g.at[b],
                                  s_g.at[b]).wait()

        def st_start(t, b):
            pltpu.async_copy(xb.at[b], out5.at[pl.ds(t * SPC, SPC), :, w],
                             s_st.at[b])

        def st_wait(t, b):
            pltpu.make_async_copy(xb.at[b], out5.at[pl.ds(t * SPC, SPC), :, w],
                                  s_st.at[b]).wait()

        rows = [jnp.arange(16, dtype=jnp.int32) + 16 * blk for blk in range(8)]

        def compute(b):
            gb = g.at[b]
            xbb = xb.at[b]

            @plsc.parallel_loop(0, 32 * SPC, unroll=8)
            def _(q):
                k = q // 32
                d = q % 32
                dcol = jnp.full((16,), d, dtype=jnp.int32)
                dh = d // 8
                off = (d % 8) * 128
                base = k * 128
                vals = [plsc.load_gather(gb, [base + rows[blk], dcol])
                        for blk in range(8)]
                for blk in range(8):
                    plsc.addupdate(
                        xbb.at[k, dh, pl.ds(off + blk * 16, 16)], vals[blk])

        for t in range(NBUF):
            ld_start(t, t)
            g_start(t, t)
        for t in (0, 1):
            g_wait(t, t)
            ld_wait(t, t)
            compute(t)
            st_start(t, t)

        @pl.loop(2, T - 2, step=4)
        def _(j):
            for k in range(4):
                t = j + k
                b = (2 + k) % 4
                bn = (b + 2) % 4
                st_wait(t - 2, bn)
                ld_start(t + 2, bn)
                g_start(t + 2, bn)
                g_wait(t, b)
                ld_wait(t, b)
                compute(b)
                st_start(t, b)

        for t in (T - 2, T - 1):
            b = t % 4
            st_wait(t - 2, (t - 2) % 4)
            g_wait(t, b)
            ld_wait(t, b)
            compute(b)
            st_start(t, b)
        st_wait(T - 2, (T - 2) % 4)
        st_wait(T - 1, (T - 1) % 4)

    return run


def kernel(inputs, labels, embedding):
    b, s, d = inputs.shape
    x5 = (inputs.transpose(1, 2, 0).reshape(s, 4, 8, 32, 128)
          .transpose(0, 1, 3, 2, 4).reshape(s, 4, 32, 1024))
    l4 = (labels.astype(jnp.int32).T.reshape(25, 8, 32, 128)
          .transpose(0, 2, 1, 3).reshape(25, 32, 1024))
    emb_rm = _table_relayout(embedding.T).reshape(V, d)
    out5 = _make_kernel()(x5, l4, emb_rm)
    return (out5.reshape(s, 4, 32, 8, 128).transpose(0, 1, 3, 2, 4)
            .reshape(s, d, b).transpose(2, 0, 1))


# speedup vs baseline: 1.4469x; 1.0270x over previous
import functools

import jax
import jax.numpy as jnp
from jax import lax
from jax.experimental import pallas as pl
from jax.experimental.pallas import tpu as pltpu
from jax.experimental.pallas import tpu_sc as plsc

NC = 2
NS = 16
NW = NC * NS
S = 200
SPC = 2
T = S // SPC
NBUF = 4

V = 1000000
LB = 8192
TC_GRID = (V + LB - 1) // LB


def _tc_relayout_body(x_ref, o_ref):
    t = x_ref[...].T.reshape(LB // 4, 4, 32)
    for j in range(4):
        o_ref[:, j * 32:(j + 1) * 32] = t[:, j, :]


def _table_relayout(emb_t):
    return pl.pallas_call(
        _tc_relayout_body,
        grid=(TC_GRID,),
        in_specs=[pl.BlockSpec((32, LB), lambda i: (0, i))],
        out_specs=pl.BlockSpec((LB // 4, 128), lambda i: (i, 0)),
        out_shape=jax.ShapeDtypeStruct((V // 4, 128), jnp.float32),
    )(emb_t)


def _make_kernel():
    mesh = plsc.VectorSubcoreMesh(core_axis_name="c", subcore_axis_name="s")

    @functools.partial(
        pl.kernel,
        out_type=jax.ShapeDtypeStruct((S, 4, NW, 1024), jnp.float32),
        mesh=mesh,
        scratch_types=[
            pltpu.VMEM((25, 1024), jnp.int32),
            pltpu.VMEM((NBUF, 128 * SPC, 32), jnp.float32),
            pltpu.VMEM((NBUF, SPC, 4, 1024), jnp.float32),
            pltpu.SemaphoreType.DMA((NBUF,)),
            pltpu.SemaphoreType.DMA((NBUF,)),
            pltpu.SemaphoreType.DMA((NBUF,)),
        ],
        compiler_params=pltpu.CompilerParams(
            use_tc_tiling_on_sc=False, needs_layout_passes=False),
    )
    def run(x5, l4, emb, out5, lv, g, xb, s_ld, s_g, s_st):
        w = lax.axis_index("s") * NC + lax.axis_index("c")
        pltpu.sync_copy(l4.at[:, w], lv)

        def idx_ref(t):
            return lv.at[t // 4, pl.ds((t % 4) * 256, 256)]

        def ld_start(t, b):
            pltpu.async_copy(x5.at[pl.ds(t * SPC, SPC), :, w], xb.at[b],
                             s_ld.at[b])

        def ld_wait(t, b):
            pltpu.make_async_copy(x5.at[pl.ds(t * SPC, SPC), :, w], xb.at[b],
                                  s_ld.at[b]).wait()

        def g_start(t, b):
            pltpu.async_copy(emb.at[idx_ref(t)], g.at[b], s_g.at[b])

        def g_wait(t, b):
            pltpu.make_async_copy(emb.at[idx_ref(t)], g.at[b],
                                  s_g.at[b]).wait()

        def st_start(t, b):
            pltpu.async_copy(xb.at[b], out5.at[pl.ds(t * SPC, SPC), :, w],
                             s_st.at[b])

        def st_wait(t, b):
            pltpu.make_async_copy(xb.at[b], out5.at[pl.ds(t * SPC, SPC), :, w],
                                  s_st.at[b]).wait()

        rows = [jnp.arange(16, dtype=jnp.int32) + 16 * blk for blk in range(8)]

        def compute(b):
            gb = g.at[b]
            xbb = xb.at[b]

            @plsc.parallel_loop(0, 32 * SPC, unroll=8)
            def _(q):
                k = q // 32
                d = q % 32
                dcol = jnp.full((16,), d, dtype=jnp.int32)
                dh = d // 8
                off = (d % 8) * 128
                base = k * 128
                vals = [plsc.load_gather(gb, [base + rows[blk], dcol])
                        for blk in range(8)]
                for blk in range(8):
                    plsc.addupdate(
                        xbb.at[k, dh, pl.ds(off + blk * 16, 16)], vals[blk])

        for t in range(NBUF):
            ld_start(t, t)
            g_start(t, t)
        for t in (0, 1):
            g_wait(t, t)
            ld_wait(t, t)
            compute(t)
            st_start(t, t)

        @pl.loop(2, T - 2, step=4)
        def _(j):
            for k in range(4):
                t = j + k
                b = (2 + k) % 4
                bn = (b + 2) % 4
                st_wait(t - 2, bn)
                ld_start(t + 2, bn)
                g_start(t + 2, bn)
                g_wait(t, b)
                ld_wait(t, b)
                compute(b)
                st_start(t, b)

        for t in (T - 2, T - 1):
            b = t % 4
            st_wait(t - 2, (t - 2) % 4)
            g_wait(t, b)
            ld_wait(t, b)
            compute(b)
            st_start(t, b)
        st_wait(T - 2, (T - 2) % 4)
        st_wait(T - 1, (T - 1) % 4)

    return run


def kernel(inputs, labels, embedding):
    b, s, d = inputs.shape
    x5 = (inputs.transpose(1, 2, 0).reshape(s, 4, 8, 32, 128)
          .transpose(0, 1, 3, 2, 4).reshape(s, 4, 32, 1024))
    l4 = (labels.astype(jnp.int32).T.reshape(25, 8, 32, 128)
          .transpose(0, 2, 1, 3).reshape(25, 32, 1024))
    emb_rm = _table_relayout(embedding.T).reshape(V, d)
    out5 = _make_kernel()(x5, l4, emb_rm)
    return (out5.reshape(s, 4, 32, 8, 128).transpose(0, 1, 3, 2, 4)
            .reshape(s, d, b).transpose(2, 0, 1))


# speedup vs baseline: 1.4672x; 1.0140x over previous
import functools

import jax
import jax.numpy as jnp
from jax import lax
from jax.experimental import pallas as pl
from jax.experimental.pallas import tpu as pltpu
from jax.experimental.pallas import tpu_sc as plsc

NC = 2
NS = 16
NW = NC * NS
S = 200
SPC = 2
T = S // SPC
NBUF = 4

V = 1000000
LB = 16384
TC_GRID = (V + LB - 1) // LB


def _tc_relayout_body(x_ref, o_ref):
    t = x_ref[...].T.reshape(LB // 4, 4, 32)
    for j in range(4):
        o_ref[:, j * 32:(j + 1) * 32] = t[:, j, :]


def _table_relayout(emb_t):
    return pl.pallas_call(
        _tc_relayout_body,
        grid=(TC_GRID,),
        in_specs=[pl.BlockSpec((32, LB), lambda i: (0, i))],
        out_specs=pl.BlockSpec((LB // 4, 128), lambda i: (i, 0)),
        out_shape=jax.ShapeDtypeStruct((V // 4, 128), jnp.float32),
    )(emb_t)


def _make_kernel():
    mesh = plsc.VectorSubcoreMesh(core_axis_name="c", subcore_axis_name="s")

    @functools.partial(
        pl.kernel,
        out_type=jax.ShapeDtypeStruct((S, 4, NW, 1024), jnp.float32),
        mesh=mesh,
        scratch_types=[
            pltpu.VMEM((25, 1024), jnp.int32),
            pltpu.VMEM((NBUF, 128 * SPC, 32), jnp.float32),
            pltpu.VMEM((NBUF, SPC, 4, 1024), jnp.float32),
            pltpu.SemaphoreType.DMA((NBUF,)),
            pltpu.SemaphoreType.DMA((NBUF,)),
            pltpu.SemaphoreType.DMA((NBUF,)),
        ],
        compiler_params=pltpu.CompilerParams(
            use_tc_tiling_on_sc=False, needs_layout_passes=False),
    )
    def run(x5, l4, emb, out5, lv, g, xb, s_ld, s_g, s_st):
        w = lax.axis_index("s") * NC + lax.axis_index("c")
        pltpu.sync_copy(l4.at[:, w], lv)

        def idx_ref(t):
            return lv.at[t // 4, pl.ds((t % 4) * 256, 256)]

        def ld_start(t, b):
            pltpu.async_copy(x5.at[pl.ds(t * SPC, SPC), :, w], xb.at[b],
                             s_ld.at[b])

        def ld_wait(t, b):
            pltpu.make_async_copy(x5.at[pl.ds(t * SPC, SPC), :, w], xb.at[b],
                                  s_ld.at[b]).wait()

        def g_start(t, b):
            pltpu.async_copy(emb.at[idx_ref(t)], g.at[b], s_g.at[b])

        def g_wait(t, b):
            pltpu.make_async_copy(emb.at[idx_ref(t)], g.at[b],
                                  s_g.at[b]).wait()

        def st_start(t, b):
            pltpu.async_copy(xb.at[b], out5.at[pl.ds(t * SPC, SPC), :, w],
                             s_st.at[b])

        def st_wait(t, b):
            pltpu.make_async_copy(xb.at[b], out5.at[pl.ds(t * SPC, SPC), :, w],
                                  s_st.at[b]).wait()

        rows = [jnp.arange(16, dtype=jnp.int32) + 16 * blk for blk in range(8)]

        def compute(b):
            gb = g.at[b]
            xbb = xb.at[b]

            @plsc.parallel_loop(0, 32 * SPC, unroll=8)
            def _(q):
                k = q // 32
                d = q % 32
                dcol = jnp.full((16,), d, dtype=jnp.int32)
                dh = d // 8
                off = (d % 8) * 128
                base = k * 128
                vals = [plsc.load_gather(gb, [base + rows[blk], dcol])
                        for blk in range(8)]
                for blk in range(8):
                    plsc.addupdate(
                        xbb.at[k, dh, pl.ds(off + blk * 16, 16)], vals[blk])

        for t in range(NBUF):
            ld_start(t, t)
            g_start(t, t)
        for t in (0, 1):
            g_wait(t, t)
            ld_wait(t, t)
            compute(t)
            st_start(t, t)

        @pl.loop(2, T - 2, step=4)
        def _(j):
            for k in range(4):
                t = j + k
                b = (2 + k) % 4
                bn = (b + 2) % 4
                st_wait(t - 2, bn)
                ld_start(t + 2, bn)
                g_start(t + 2, bn)
                g_wait(t, b)
                ld_wait(t, b)
                compute(b)
                st_start(t, b)

        for t in (T - 2, T - 1):
            b = t % 4
            st_wait(t - 2, (t - 2) % 4)
            g_wait(t, b)
            ld_wait(t, b)
            compute(b)
            st_start(t, b)
        st_wait(T - 2, (T - 2) % 4)
        st_wait(T - 1, (T - 1) % 4)

    return run


def kernel(inputs, labels, embedding):
    b, s, d = inputs.shape
    x5 = (inputs.transpose(1, 2, 0).reshape(s, 4, 8, 32, 128)
          .transpose(0, 1, 3, 2, 4).reshape(s, 4, 32, 1024))
    l4 = (labels.astype(jnp.int32).T.reshape(25, 8, 32, 128)
          .transpose(0, 2, 1, 3).reshape(25, 32, 1024))
    emb_rm = _table_relayout(embedding.T).reshape(V, d)
    out5 = _make_kernel()(x5, l4, emb_rm)
    return (out5.reshape(s, 4, 32, 8, 128).transpose(0, 1, 3, 2, 4)
            .reshape(s, d, b).transpose(2, 0, 1))
